# baseline (speedup 1.0000x reference)
import jax
import jax.numpy as jnp
from jax.experimental import pallas as pl

_K = 8192
_D = 256
_N = 8192
_TB = 512
_KC = 1024
_COMMIT = 0.25


def _vq_body(zn_ref, wn_ref, z_ref, w_ref, zq_ref, idx_ref, loss_ref):
    z = z_ref[...]
    zn = zn_ref[...]
    nsteps = _K // _KC

    def dist_step(c, carry):
        bmin, bidx = carry
        wc = w_ref[pl.ds(c * _KC, _KC), :]
        wn = wn_ref[:, pl.ds(c * _KC, _KC)]
        m = jax.lax.dot_general(z, wc, (((1,), (1,)), ((), ())),
                                preferred_element_type=jnp.float32)
        s = (zn - 2.0 * m) + wn
        rmin = jnp.min(s, axis=1, keepdims=True)
        kio = jax.lax.broadcasted_iota(jnp.int32, (_TB, _KC), 1) + c * _KC
        cand = jnp.min(jnp.where(s == rmin, kio, jnp.int32(2**30)),
                       axis=1, keepdims=True)
        take = rmin < bmin
        return (jnp.where(take, rmin, bmin), jnp.where(take, cand, bidx))

    init = (jnp.full((_TB, 1), jnp.inf, jnp.float32),
            jnp.full((_TB, 1), 2**30, jnp.int32))
    _, idx = jax.lax.fori_loop(0, nsteps, dist_step, init)
    idx_ref[...] = idx

    def gather_step(c, acc):
        wc = w_ref[pl.ds(c * _KC, _KC), :]
        kio = jax.lax.broadcasted_iota(jnp.int32, (_TB, _KC), 1) + c * _KC
        oh = (idx == kio).astype(jnp.float32)
        return acc + jax.lax.dot_general(oh, wc, (((1,), (0,)), ((), ())),
                                         preferred_element_type=jnp.float32)

    zq = jax.lax.fori_loop(0, nsteps, gather_step,
                           jnp.zeros((_TB, _D), jnp.float32))
    zq_ref[...] = zq

    part = jnp.sum((z - zq) ** 2).reshape(1, 1)

    @pl.when(pl.program_id(0) == 0)
    def _():
        loss_ref[...] = part

    @pl.when(pl.program_id(0) != 0)
    def _():
        loss_ref[...] = loss_ref[...] + part


def kernel(z_e, weight):
    B, D, H, W = z_e.shape
    z_flat = jnp.transpose(z_e, (0, 2, 3, 1)).reshape(-1, D)
    zn = jnp.sum(z_flat ** 2, axis=1, keepdims=True)
    wn = jnp.sum(weight ** 2, axis=1, keepdims=True).T

    zq_flat, idx, loss_part = pl.pallas_call(
        _vq_body,
        grid=(_N // _TB,),
        in_specs=[
            pl.BlockSpec((_TB, 1), lambda i: (i, 0)),
            pl.BlockSpec((1, _K), lambda i: (0, 0)),
            pl.BlockSpec((_TB, _D), lambda i: (i, 0)),
            pl.BlockSpec((_K, _D), lambda i: (0, 0)),
        ],
        out_specs=[
            pl.BlockSpec((_TB, _D), lambda i: (i, 0)),
            pl.BlockSpec((_TB, 1), lambda i: (i, 0)),
            pl.BlockSpec((1, 1), lambda i: (0, 0)),
        ],
        out_shape=[
            jax.ShapeDtypeStruct((_N, _D), jnp.float32),
            jax.ShapeDtypeStruct((_N, 1), jnp.int32),
            jax.ShapeDtypeStruct((1, 1), jnp.float32),
        ],
    )(zn, wn, z_flat, weight)

    commit_loss = loss_part[0, 0] / (_N * _D)
    zq_sp = jnp.transpose(zq_flat.reshape(B, H, W, D), (0, 3, 1, 2))
    z_q = z_e + jax.lax.stop_gradient(zq_sp - z_e)
    return (z_q, _COMMIT * commit_loss, idx.reshape(B, H, W))
